# baseline (speedup 1.0000x reference)
import functools

import jax
import jax.numpy as jnp
from jax import lax
from jax.experimental import pallas as pl
from jax.experimental.pallas import tpu as pltpu


def _xattn_kernel(q_ref, k_ref, v_ref,
                  wq_ref, bq_ref, wk_ref, bk_ref, wv_ref, bv_ref,
                  wo_ref, bo_ref,
                  o_ref,
                  kp_ref, vp_ref, ctx_ref,
                  *, num_heads, head_dim):
    @pl.when(pl.program_id(1) == 0)
    def _():
        kb = k_ref[0].astype(jnp.bfloat16)
        vb = v_ref[0].astype(jnp.bfloat16)
        K = jnp.dot(kb, wk_ref[...],
                    preferred_element_type=jnp.float32) + bk_ref[...]
        V = jnp.dot(vb, wv_ref[...],
                    preferred_element_type=jnp.float32) + bv_ref[...]
        kp_ref[...] = K.astype(jnp.bfloat16)
        vp_ref[...] = V.astype(jnp.bfloat16)

    qb = q_ref[0].astype(jnp.bfloat16)
    Q = (jnp.dot(qb, wq_ref[...],
                 preferred_element_type=jnp.float32)
         + bq_ref[...]).astype(jnp.bfloat16)

    dn_qk = (((1,), (1,)), ((), ()))

    for h in range(num_heads):
        lo = h * head_dim
        hi = lo + head_dim
        Qh = Q[:, lo:hi]
        Kh = kp_ref[:, lo:hi]
        Vh = vp_ref[:, lo:hi]

        s = lax.dot_general(Qh, Kh, dn_qk,
                            preferred_element_type=jnp.float32)

        m = jnp.max(s, axis=-1, keepdims=True)
        e = jnp.exp(s - m)
        inv = pl.reciprocal(jnp.sum(e, axis=-1, keepdims=True), approx=True)
        ch = jnp.dot(e.astype(jnp.bfloat16), Vh,
                     preferred_element_type=jnp.float32)
        ctx_ref[:, lo:hi] = (ch * inv).astype(jnp.bfloat16)

    out = jnp.dot(ctx_ref[...], wo_ref[...],
                  preferred_element_type=jnp.float32) + bo_ref[...]
    o_ref[0] = out.astype(o_ref.dtype)


def kernel(query, key, value, wq, bq, wk, bk, wv, bv, wo, bo):
    B, Lq, hidden = query.shape
    _, Lk, k_dim = key.shape
    _, _, v_dim = value.shape
    num_heads = 8
    head_dim = hidden // num_heads

    scale = jnp.asarray(1.0 / (head_dim ** 0.5), jnp.float32)
    wq_b = (wq * scale).astype(jnp.bfloat16)
    bq_s = bq * scale
    wk_b = wk.astype(jnp.bfloat16)
    wv_b = wv.astype(jnp.bfloat16)
    wo_b = wo.astype(jnp.bfloat16)

    q_tile = 128 if Lq % 128 == 0 else Lq
    nq = Lq // q_tile

    kfn = functools.partial(_xattn_kernel,
                            num_heads=num_heads, head_dim=head_dim)

    def resident(shape):
        return pl.BlockSpec(shape, lambda b, i: (0, 0))

    grid_spec = pltpu.PrefetchScalarGridSpec(
        num_scalar_prefetch=0,
        grid=(B, nq),
        in_specs=[
            pl.BlockSpec((1, q_tile, hidden), lambda b, i: (b, i, 0)),
            pl.BlockSpec((1, Lk, k_dim), lambda b, i: (b, 0, 0)),
            pl.BlockSpec((1, Lk, v_dim), lambda b, i: (b, 0, 0)),
            resident((hidden, hidden)), resident((1, hidden)),
            resident((k_dim, hidden)), resident((1, hidden)),
            resident((v_dim, hidden)), resident((1, hidden)),
            resident((hidden, hidden)), resident((1, hidden)),
        ],
        out_specs=pl.BlockSpec((1, q_tile, hidden), lambda b, i: (b, i, 0)),
        scratch_shapes=[
            pltpu.VMEM((Lk, hidden), jnp.bfloat16),
            pltpu.VMEM((Lk, hidden), jnp.bfloat16),
            pltpu.VMEM((q_tile, hidden), jnp.bfloat16),
        ],
    )

    return pl.pallas_call(
        kfn,
        out_shape=jax.ShapeDtypeStruct((B, Lq, hidden), query.dtype),
        grid_spec=grid_spec,
        compiler_params=pltpu.CompilerParams(
            dimension_semantics=("parallel", "arbitrary"),
        ),
    )(query, key, value, wq_b, bq_s, wk_b, bk, wv_b, bv, wo_b, bo)
